# baseline (speedup 1.0000x reference)
import functools

import jax
import jax.numpy as jnp
from jax import lax
from jax.experimental import pallas as pl
from jax.experimental.pallas import tpu as pltpu
from jax.experimental.pallas import tpu_sc as plsc

D_MODEL = 64
EPS_FOLDED = 1e-5 / D_MODEL
LANES = 16
NUM_WORKERS = 32
GATHER_BATCH = 128
L_PER_CHUNK = 1
CHUNK = L_PER_CHUNK * GATHER_BATCH
IDX_STAGE_ROWS = 32


def _rsqrt(x):
  i = plsc.bitcast(x, jnp.int32)
  i = jnp.int32(0x5F3759DF) - (i >> 1)
  y = plsc.bitcast(i, jnp.float32)
  for _ in range(2):
    y = y * (1.5 - 0.5 * x * y * y)
  return y


def _ln_kernel(inp_hbm, table_hbm, out_hbm, idx_stage, idx_t, rows0, rows1,
               rows2, rows3, stg0, stg1, scr_p, scr_q, scr_m, scr_r, ipad,
               sg0, sg1, sg2, sg3, so0, so1):
  rows_v = (rows0, rows1, rows2, rows3)
  stg = (stg0, stg1)
  sg = (sg0, sg1, sg2, sg3)
  so = (so0, so1)
  b_total, l_total = inp_hbm.shape
  wid = lax.axis_index("s") * 2 + lax.axis_index("c")
  b_block = b_total // NUM_WORKERS
  b0 = wid * b_block
  num_chunks = l_total // L_PER_CHUNK
  lane_iota = lax.iota(jnp.int32, LANES)

  n_col_blocks = l_total // LANES
  col_tail = l_total - n_col_blocks * LANES
  for h in range(b_block // IDX_STAGE_ROWS):
    pltpu.sync_copy(
        inp_hbm.at[pl.ds(pl.multiple_of(b0 + h * IDX_STAGE_ROWS, 8),
                         IDX_STAGE_ROWS)],
        idx_stage,
    )

    def tr_block(rblk, col0, c_lo):
      r0 = rblk * LANES
      for r in range(LANES):
        ipad[r, pl.ds(0, LANES)] = idx_stage[r0 + r, pl.ds(col0, LANES)]
      for c in range(c_lo, LANES):
        colv = plsc.load_gather(
            ipad, [lane_iota, jnp.full((LANES,), c, jnp.int32)]
        )
        idx_t[col0 + c, pl.ds(h * IDX_STAGE_ROWS + r0, LANES)] = colv

    def tr_body(rblk, _):
      for cblk in range(n_col_blocks):
        tr_block(rblk, cblk * LANES, 0)
      if col_tail:
        tr_block(rblk, l_total - LANES, LANES - col_tail)
      return 0

    lax.fori_loop(0, IDX_STAGE_ROWS // LANES, tr_body, 0)

  def fire_gathers(c, b):
    pltpu.async_copy(
        table_hbm.at[idx_t.at[c]],
        rows_v[b],
        sg[b],
    )

  def drain_gathers(c, b):
    pltpu.make_async_copy(
        table_hbm.at[idx_t.at[c]],
        rows_v[b],
        sg[b],
    ).wait()

  half_block = GATHER_BATCH // 2

  def fire_out(c, b):
    for q in range(L_PER_CHUNK):
      pltpu.async_copy(
          stg[b].at[pl.ds(q * half_block, half_block)],
          out_hbm.at[c * L_PER_CHUNK + q,
                     pl.ds(pl.multiple_of(wid * half_block, 8), half_block)],
          so[b],
      )

  def drain_out(b):
    for q in range(L_PER_CHUNK):
      pltpu.make_async_copy(
          stg[b].at[pl.ds(q * half_block, half_block)],
          out_hbm.at[q, pl.ds(pl.multiple_of(wid * half_block, 8), half_block)],
          so[b],
      ).wait()

  def compute(c, b, bs):
    buf = rows_v[b]
    nq = D_MODEL // LANES

    def one_group(g):
      row0 = g * LANES
      for r in range(LANES):
        vs = [buf[row0 + r, pl.ds(k * LANES, LANES)] for k in range(nq)]
        p = vs[0] + vs[1] + vs[2] + vs[3]
        q = (vs[0] * vs[0] + vs[1] * vs[1]) + (vs[2] * vs[2]
                                               + vs[3] * vs[3])
        scr_p[r, pl.ds(0, LANES)] = p
        scr_q[r, pl.ds(0, LANES)] = q
      sums = jnp.zeros((LANES,), jnp.float32)
      sumsq = jnp.zeros((LANES,), jnp.float32)
      for k in range(LANES):
        kv = jnp.full((LANES,), k, jnp.int32)
        sums = sums + plsc.load_gather(scr_p, [lane_iota, kv])
        sumsq = sumsq + plsc.load_gather(scr_q, [lane_iota, kv])
      mean = sums * (1.0 / D_MODEL)
      var = sumsq * (1.0 / D_MODEL) - mean * mean
      rstd = _rsqrt(var + EPS_FOLDED)
      for k in range(LANES):
        scr_m[k, pl.ds(0, LANES)] = mean
        scr_r[k, pl.ds(0, LANES)] = rstd
      for r in range(LANES):
        rv = jnp.full((LANES,), r, jnp.int32)
        sm = plsc.load_gather(scr_m, [lane_iota, rv])
        sr = plsc.load_gather(scr_r, [lane_iota, rv])
        for k in range(nq):
          w = buf[row0 + r, pl.ds(k * LANES, LANES)]
          stg[bs][g * (LANES // 2) + r // 2,
                  pl.ds((r % 2) * D_MODEL + k * LANES, LANES)] = (
                      (w - sm) * sr)

    def group_body(g, _):
      one_group(g)
      return 0

    lax.fori_loop(0, CHUNK // LANES, group_body, 0)

  fire_gathers(0, 0)
  fire_gathers(1, 1)
  fire_gathers(2, 2)

  def quad_body(p, _):
    for m in range(4):
      c = 4 * p + m

      @pl.when(c + 3 < num_chunks)
      def _():
        fire_gathers(c + 3, (m + 3) % 4)

      drain_gathers(c, m)

      @pl.when(c >= 2)
      def _():
        drain_out(m % 2)

      compute(c, m, m % 2)
      fire_out(c, m % 2)
    return 0

  lax.fori_loop(0, num_chunks // 4, quad_body, 0)
  drain_out(0)
  drain_out(1)


def kernel(inp, emb_weight):
  b, l = inp.shape
  grid_kernel = functools.partial(
      pl.kernel,
      out_type=jax.ShapeDtypeStruct((l, b // 2, 2 * D_MODEL), jnp.float32),
      mesh=plsc.VectorSubcoreMesh(core_axis_name="c", subcore_axis_name="s"),
      compiler_params=pltpu.CompilerParams(
          needs_layout_passes=False, use_tc_tiling_on_sc=False
      ),
      scratch_types=[
          pltpu.VMEM((IDX_STAGE_ROWS, l), jnp.int32),
          pltpu.VMEM((l, b // NUM_WORKERS), jnp.int32),
          pltpu.VMEM((CHUNK, 2 * D_MODEL), jnp.float32),
          pltpu.VMEM((CHUNK, 2 * D_MODEL), jnp.float32),
          pltpu.VMEM((CHUNK, 2 * D_MODEL), jnp.float32),
          pltpu.VMEM((CHUNK, 2 * D_MODEL), jnp.float32),
          pltpu.VMEM((CHUNK // 2, 2 * D_MODEL), jnp.float32),
          pltpu.VMEM((CHUNK // 2, 2 * D_MODEL), jnp.float32),
          pltpu.VMEM((LANES, LANES + 1), jnp.float32),
          pltpu.VMEM((LANES, LANES + 1), jnp.float32),
          pltpu.VMEM((LANES, LANES + 1), jnp.float32),
          pltpu.VMEM((LANES, LANES + 1), jnp.float32),
          pltpu.VMEM((LANES, LANES + 1), jnp.int32),
          pltpu.SemaphoreType.DMA,
          pltpu.SemaphoreType.DMA,
          pltpu.SemaphoreType.DMA,
          pltpu.SemaphoreType.DMA,
          pltpu.SemaphoreType.DMA,
          pltpu.SemaphoreType.DMA,
      ],
  )
  table_padded = jnp.pad(emb_weight, ((0, 0), (0, D_MODEL)))
  out = grid_kernel(_ln_kernel)(inp.astype(jnp.int32), table_padded)
  return out.reshape(l, b, D_MODEL)


# speedup vs baseline: 1.0093x; 1.0093x over previous
import functools

import jax
import jax.numpy as jnp
from jax import lax
from jax.experimental import pallas as pl
from jax.experimental.pallas import tpu as pltpu
from jax.experimental.pallas import tpu_sc as plsc

D_MODEL = 64
EPS_FOLDED = 1e-5 / D_MODEL
LANES = 16
NUM_WORKERS = 32
GATHER_BATCH = 128
L_PER_CHUNK = 2
CHUNK = L_PER_CHUNK * GATHER_BATCH
IDX_STAGE_ROWS = 16


def _rsqrt(x):
  i = plsc.bitcast(x, jnp.int32)
  i = jnp.int32(0x5F3759DF) - (i >> 1)
  y = plsc.bitcast(i, jnp.float32)
  for _ in range(2):
    y = y * (1.5 - 0.5 * x * y * y)
  return y


def _ln_kernel(inp_hbm, table_hbm, out_hbm, idx_stage, idx_t, rows0, rows1,
               stg0, stg1, scr_p, scr_q, scr_m, scr_r, ipad, sg0, sg1, so0,
               so1):
  rows_v = (rows0, rows1)
  stg = (stg0, stg1)
  sg = (sg0, sg1)
  so = (so0, so1)
  b_total, l_total = inp_hbm.shape
  wid = lax.axis_index("s") * 2 + lax.axis_index("c")
  b_block = b_total // NUM_WORKERS
  b0 = wid * b_block
  num_chunks = l_total // L_PER_CHUNK
  lane_iota = lax.iota(jnp.int32, LANES)

  n_col_blocks = l_total // LANES
  col_tail = l_total - n_col_blocks * LANES
  for h in range(b_block // IDX_STAGE_ROWS):
    pltpu.sync_copy(
        inp_hbm.at[pl.ds(pl.multiple_of(b0 + h * IDX_STAGE_ROWS, 8),
                         IDX_STAGE_ROWS)],
        idx_stage,
    )

    def tr_block(rblk, col0, c_lo):
      r0 = rblk * LANES
      for r in range(LANES):
        ipad[r, pl.ds(0, LANES)] = idx_stage[r0 + r, pl.ds(col0, LANES)]
      for c in range(c_lo, LANES):
        colv = plsc.load_gather(
            ipad, [lane_iota, jnp.full((LANES,), c, jnp.int32)]
        )
        idx_t[col0 + c, pl.ds(h * IDX_STAGE_ROWS + r0, LANES)] = colv

    def tr_body(rblk, _):
      for cblk in range(n_col_blocks):
        tr_block(rblk, cblk * LANES, 0)
      if col_tail:
        tr_block(rblk, l_total - LANES, LANES - col_tail)
      return 0

    lax.fori_loop(0, IDX_STAGE_ROWS // LANES, tr_body, 0)

  def fire_gathers(c, b):
    for q in range(L_PER_CHUNK):
      pltpu.async_copy(
          table_hbm.at[idx_t.at[c * L_PER_CHUNK + q]],
          rows_v[b].at[pl.ds(q * GATHER_BATCH, GATHER_BATCH)],
          sg[b],
      )

  def drain_gathers(c, b):
    for q in range(L_PER_CHUNK):
      pltpu.make_async_copy(
          table_hbm.at[idx_t.at[c * L_PER_CHUNK + q]],
          rows_v[b].at[pl.ds(q * GATHER_BATCH, GATHER_BATCH)],
          sg[b],
      ).wait()

  half_block = GATHER_BATCH // 2

  def fire_out(c, b):
    for q in range(L_PER_CHUNK):
      pltpu.async_copy(
          stg[b].at[pl.ds(q * half_block, half_block)],
          out_hbm.at[c * L_PER_CHUNK + q,
                     pl.ds(pl.multiple_of(wid * half_block, 8), half_block)],
          so[b],
      )

  def drain_out(b):
    for q in range(L_PER_CHUNK):
      pltpu.make_async_copy(
          stg[b].at[pl.ds(q * half_block, half_block)],
          out_hbm.at[q, pl.ds(pl.multiple_of(wid * half_block, 8), half_block)],
          so[b],
      ).wait()

  def compute(c, b, bs):
    buf = rows_v[b]
    nq = D_MODEL // LANES

    def one_group(g):
      row0 = g * LANES
      for r in range(LANES):
        vs = [buf[row0 + r, pl.ds(k * LANES, LANES)] for k in range(nq)]
        p = vs[0] + vs[1] + vs[2] + vs[3]
        q = (vs[0] * vs[0] + vs[1] * vs[1]) + (vs[2] * vs[2]
                                               + vs[3] * vs[3])
        scr_p[r, pl.ds(0, LANES)] = p
        scr_q[r, pl.ds(0, LANES)] = q
      sums = jnp.zeros((LANES,), jnp.float32)
      sumsq = jnp.zeros((LANES,), jnp.float32)
      for k in range(LANES):
        kv = jnp.full((LANES,), k, jnp.int32)
        sums = sums + plsc.load_gather(scr_p, [lane_iota, kv])
        sumsq = sumsq + plsc.load_gather(scr_q, [lane_iota, kv])
      mean = sums * (1.0 / D_MODEL)
      var = sumsq * (1.0 / D_MODEL) - mean * mean
      rstd = _rsqrt(var + EPS_FOLDED)
      for k in range(LANES):
        scr_m[k, pl.ds(0, LANES)] = mean
        scr_r[k, pl.ds(0, LANES)] = rstd
      for r in range(LANES):
        rv = jnp.full((LANES,), r, jnp.int32)
        sm = plsc.load_gather(scr_m, [lane_iota, rv])
        sr = plsc.load_gather(scr_r, [lane_iota, rv])
        for k in range(nq):
          w = buf[row0 + r, pl.ds(k * LANES, LANES)]
          stg[bs][g * (LANES // 2) + r // 2,
                  pl.ds((r % 2) * D_MODEL + k * LANES, LANES)] = (
                      (w - sm) * sr)

    def group_body(g, _):
      one_group(g)
      return 0

    lax.fori_loop(0, CHUNK // LANES, group_body, 0)

  fire_gathers(0, 0)
  fire_gathers(1, 1)

  def pair_body(p, _):
    for m in range(2):
      c = 2 * p + m
      drain_gathers(c, m)

      @pl.when(c >= 2)
      def _():
        drain_out(m)

      compute(c, m, m)
      fire_out(c, m)

      @pl.when(c + 2 < num_chunks)
      def _():
        fire_gathers(c + 2, m)
    return 0

  lax.fori_loop(0, num_chunks // 2, pair_body, 0)
  drain_out(0)
  drain_out(1)


def kernel(inp, emb_weight):
  b, l = inp.shape
  grid_kernel = functools.partial(
      pl.kernel,
      out_type=jax.ShapeDtypeStruct((l, b // 2, 2 * D_MODEL), jnp.float32),
      mesh=plsc.VectorSubcoreMesh(core_axis_name="c", subcore_axis_name="s"),
      compiler_params=pltpu.CompilerParams(
          needs_layout_passes=False, use_tc_tiling_on_sc=False
      ),
      scratch_types=[
          pltpu.VMEM((IDX_STAGE_ROWS, l), jnp.int32),
          pltpu.VMEM((l, b // NUM_WORKERS), jnp.int32),
          pltpu.VMEM((CHUNK, 2 * D_MODEL), jnp.float32),
          pltpu.VMEM((CHUNK, 2 * D_MODEL), jnp.float32),
          pltpu.VMEM((CHUNK // 2, 2 * D_MODEL), jnp.float32),
          pltpu.VMEM((CHUNK // 2, 2 * D_MODEL), jnp.float32),
          pltpu.VMEM((LANES, LANES + 1), jnp.float32),
          pltpu.VMEM((LANES, LANES + 1), jnp.float32),
          pltpu.VMEM((LANES, LANES + 1), jnp.float32),
          pltpu.VMEM((LANES, LANES + 1), jnp.float32),
          pltpu.VMEM((LANES, LANES + 1), jnp.int32),
          pltpu.SemaphoreType.DMA,
          pltpu.SemaphoreType.DMA,
          pltpu.SemaphoreType.DMA,
          pltpu.SemaphoreType.DMA,
      ],
  )
  table_padded = jnp.pad(emb_weight, ((0, 0), (0, D_MODEL)))
  out = grid_kernel(_ln_kernel)(inp.astype(jnp.int32), table_padded)
  return out.reshape(l, b, D_MODEL)
